# baseline (speedup 1.0000x reference)
import functools
import math

import jax
import jax.numpy as jnp
from jax import lax
from jax.experimental import pallas as pl
from jax.experimental.pallas import tpu as pltpu
from jax.experimental.pallas import tpu_sc as plsc

D_MODEL = 1024
LANES = 16
NUM_WORKERS = 32
CHUNK = 8
NBUF = 5


def _sc_embed(idx_flat, table, pe, s_len, batch):
    n = s_len * batch
    n_per_w = n // NUM_WORKERS
    nchunks = n_per_w // CHUNK
    pe_chunk = CHUNK // batch
    scale = jnp.float32(math.sqrt(D_MODEL))
    mesh = plsc.VectorSubcoreMesh(core_axis_name="c", subcore_axis_name="s")

    @functools.partial(
        pl.kernel,
        mesh=mesh,
        out_type=jax.ShapeDtypeStruct((s_len, batch, D_MODEL), jnp.float32),
        scratch_types=[pltpu.VMEM((n_per_w,), jnp.int32)]
        + [pltpu.VMEM((CHUNK, D_MODEL), jnp.float32)] * NBUF
        + [pltpu.VMEM((pe_chunk, batch, D_MODEL), jnp.float32)] * NBUF
        + [pltpu.VMEM((pe_chunk, 1, D_MODEL), jnp.float32)] * NBUF
        + [pltpu.SemaphoreType.DMA] * (3 * NBUF),
    )
    def k(tbl_hbm, idx_hbm, pe_hbm, out_hbm, idx_v, *scratch):
        ins = scratch[:NBUF]
        outs = scratch[NBUF:2 * NBUF]
        pes = scratch[2 * NBUF:3 * NBUF]
        gsems = scratch[3 * NBUF:4 * NBUF]
        ssems = scratch[4 * NBUF:5 * NBUF]
        psems = scratch[5 * NBUF:6 * NBUF]

        wid = lax.axis_index("s") * 2 + lax.axis_index("c")
        base = wid * n_per_w
        sbase = base // batch
        pltpu.sync_copy(idx_hbm.at[pl.ds(base, n_per_w)], idx_v)

        def issue_gather(g, p):
            off = pl.multiple_of(g * CHUNK, CHUNK)
            pltpu.async_copy(
                tbl_hbm.at[idx_v.at[pl.ds(off, CHUNK)]], ins[p], gsems[p]
            )
            pe_off = pl.multiple_of(sbase + g * pe_chunk, pe_chunk)
            pltpu.async_copy(
                pe_hbm.at[pl.ds(pe_off, pe_chunk)], pes[p], psems[p]
            )

        def wait_gather(g, p):
            off = pl.multiple_of(g * CHUNK, CHUNK)
            pltpu.make_async_copy(
                tbl_hbm.at[idx_v.at[pl.ds(off, CHUNK)]], ins[p], gsems[p]
            ).wait()
            pe_off = pl.multiple_of(sbase + g * pe_chunk, pe_chunk)
            pltpu.make_async_copy(
                pe_hbm.at[pl.ds(pe_off, pe_chunk)], pes[p], psems[p]
            ).wait()

        def issue_store(g, p):
            s0 = pl.multiple_of(sbase + g * pe_chunk, pe_chunk)
            pltpu.async_copy(outs[p], out_hbm.at[pl.ds(s0, pe_chunk)], ssems[p])

        def wait_store(g, p):
            s0 = pl.multiple_of(sbase + g * pe_chunk, pe_chunk)
            pltpu.make_async_copy(
                outs[p], out_hbm.at[pl.ds(s0, pe_chunk)], ssems[p]
            ).wait()

        def compute(p):
            @plsc.parallel_loop(0, D_MODEL, step=LANES, unroll=8)
            def _(c):
                for srow in range(pe_chunk):
                    pev = pes[p].at[
                        pl.ds(srow, 1), pl.ds(0, 1), pl.ds(c, LANES)
                    ][...]
                    for b in range(batch):
                        src = (pl.ds(srow * batch + b, 1), pl.ds(c, LANES))
                        dst = (pl.ds(srow, 1), pl.ds(b, 1), pl.ds(c, LANES))
                        outs[p].at[dst][...] = (
                            ins[p].at[src][...] * scale
                        ).reshape(1, 1, LANES) + pev

        for g in range(NBUF):
            issue_gather(g, g)

        @pl.loop(0, nchunks + NBUF - (nchunks % NBUF or NBUF), step=NBUF)
        def _(g0):
            for p in range(NBUF):
                g = g0 + p

                @pl.when(g < nchunks)
                def _():
                    @pl.when(g >= NBUF)
                    def _():
                        wait_store(g - NBUF, p)

                    wait_gather(g, p)
                    compute(p)
                    issue_store(g, p)

                    @pl.when(g + NBUF < nchunks)
                    def _():
                        issue_gather(g + NBUF, p)

        for g in range(nchunks - NBUF, nchunks):
            wait_store(g, g % NBUF)

    return k(table, idx_flat, pe)


@jax.jit
def kernel(x, table, pe):
    s_len, batch = x.shape
    idx_flat = x.reshape(s_len * batch)
    return _sc_embed(idx_flat, table, pe, s_len, batch)


# speedup vs baseline: 1.0049x; 1.0049x over previous
import functools
import math

import jax
import jax.numpy as jnp
from jax import lax
from jax.experimental import pallas as pl
from jax.experimental.pallas import tpu as pltpu
from jax.experimental.pallas import tpu_sc as plsc

D_MODEL = 1024
LANES = 16
NUM_WORKERS = 32
CHUNK = 16
NBUF = 3


def _sc_embed(idx_flat, table, pe, s_len, batch):
    n = s_len * batch
    n_per_w = n // NUM_WORKERS
    nchunks = n_per_w // CHUNK
    pe_chunk = CHUNK // batch
    scale = jnp.float32(math.sqrt(D_MODEL))
    mesh = plsc.VectorSubcoreMesh(core_axis_name="c", subcore_axis_name="s")

    @functools.partial(
        pl.kernel,
        mesh=mesh,
        out_type=jax.ShapeDtypeStruct((s_len, batch, D_MODEL), jnp.float32),
        scratch_types=[pltpu.VMEM((n_per_w,), jnp.int32)]
        + [pltpu.VMEM((CHUNK, D_MODEL), jnp.float32)] * NBUF
        + [pltpu.VMEM((pe_chunk, batch, D_MODEL), jnp.float32)] * NBUF
        + [pltpu.VMEM((pe_chunk, 1, D_MODEL), jnp.float32)] * NBUF
        + [pltpu.SemaphoreType.DMA] * (3 * NBUF),
    )
    def k(tbl_hbm, idx_hbm, pe_hbm, out_hbm, idx_v, *scratch):
        ins = scratch[:NBUF]
        outs = scratch[NBUF:2 * NBUF]
        pes = scratch[2 * NBUF:3 * NBUF]
        gsems = scratch[3 * NBUF:4 * NBUF]
        ssems = scratch[4 * NBUF:5 * NBUF]
        psems = scratch[5 * NBUF:6 * NBUF]

        wid = lax.axis_index("s") * 2 + lax.axis_index("c")
        base = wid * n_per_w
        sbase = base // batch
        pltpu.sync_copy(idx_hbm.at[pl.ds(base, n_per_w)], idx_v)

        def issue_gather(g, p):
            off = pl.multiple_of(g * CHUNK, CHUNK)
            pltpu.async_copy(
                tbl_hbm.at[idx_v.at[pl.ds(off, CHUNK)]], ins[p], gsems[p]
            )
            pe_off = pl.multiple_of(sbase + g * pe_chunk, pe_chunk)
            pltpu.async_copy(
                pe_hbm.at[pl.ds(pe_off, pe_chunk)], pes[p], psems[p]
            )

        def wait_gather(g, p):
            off = pl.multiple_of(g * CHUNK, CHUNK)
            pltpu.make_async_copy(
                tbl_hbm.at[idx_v.at[pl.ds(off, CHUNK)]], ins[p], gsems[p]
            ).wait()
            pe_off = pl.multiple_of(sbase + g * pe_chunk, pe_chunk)
            pltpu.make_async_copy(
                pe_hbm.at[pl.ds(pe_off, pe_chunk)], pes[p], psems[p]
            ).wait()

        def issue_store(g, p):
            s0 = pl.multiple_of(sbase + g * pe_chunk, pe_chunk)
            pltpu.async_copy(outs[p], out_hbm.at[pl.ds(s0, pe_chunk)], ssems[p])

        def wait_store(g, p):
            s0 = pl.multiple_of(sbase + g * pe_chunk, pe_chunk)
            pltpu.make_async_copy(
                outs[p], out_hbm.at[pl.ds(s0, pe_chunk)], ssems[p]
            ).wait()

        def compute(p):
            @plsc.parallel_loop(0, D_MODEL, step=LANES, unroll=8)
            def _(c):
                for srow in range(pe_chunk):
                    pev = pes[p].at[
                        pl.ds(srow, 1), pl.ds(0, 1), pl.ds(c, LANES)
                    ][...]
                    for b in range(batch):
                        src = (pl.ds(srow * batch + b, 1), pl.ds(c, LANES))
                        dst = (pl.ds(srow, 1), pl.ds(b, 1), pl.ds(c, LANES))
                        outs[p].at[dst][...] = (
                            ins[p].at[src][...] * scale
                        ).reshape(1, 1, LANES) + pev

        for g in range(NBUF):
            issue_gather(g, g)

        @pl.loop(0, nchunks + NBUF - (nchunks % NBUF or NBUF), step=NBUF)
        def _(g0):
            for p in range(NBUF):
                g = g0 + p

                @pl.when(g < nchunks)
                def _():
                    @pl.when(g >= NBUF)
                    def _():
                        wait_store(g - NBUF, p)

                    wait_gather(g, p)
                    compute(p)
                    issue_store(g, p)

                    @pl.when(g + NBUF < nchunks)
                    def _():
                        issue_gather(g + NBUF, p)

        for g in range(nchunks - NBUF, nchunks):
            wait_store(g, g % NBUF)

    return k(table, idx_flat, pe)


@jax.jit
def kernel(x, table, pe):
    s_len, batch = x.shape
    idx_flat = x.reshape(s_len * batch)
    return _sc_embed(idx_flat, table, pe, s_len, batch)
